# baseline (speedup 1.0000x reference)
import functools

import jax
import jax.numpy as jnp
from jax import lax
from jax.experimental import pallas as pl
from jax.experimental.pallas import tpu as pltpu
from jax.experimental.pallas import tpu_sc as plsc

B = 16384
D = 32
NC = 2
NS = 16
L = 16
NW = NC * NS
BPW = B // NW

NROWS = 1000000
NMAIN = 999936
NTAIL = NROWS - NMAIN
CPAD = 1000064
FLAT = D * CPAD


def _detile_body(a_ref, b_ref, c_ref, d_ref, tail_ref,
                 oa_ref, ob_ref, oc_ref, od_ref, sem):
    handles = []
    for t, (src, dst) in enumerate(((a_ref, oa_ref), (b_ref, ob_ref),
                                    (c_ref, oc_ref), (d_ref, od_ref))):
        for c in range(D):
            handles.append(pltpu.async_copy(
                src.at[c, pl.ds(0, NMAIN)],
                dst.at[pl.ds(c * CPAD, NMAIN)], sem))
            handles.append(pltpu.async_copy(
                tail_ref.at[pl.ds((t * D + c) * 128, 128)],
                dst.at[pl.ds(c * CPAD + NMAIN, 128)], sem))
    for h in handles:
        h.wait()


def _detile(a_t, b_t, c_t, d_t, tail_flat):
    any_spec = pl.BlockSpec(memory_space=pl.ANY)
    return pl.pallas_call(
        _detile_body,
        in_specs=[any_spec] * 5,
        out_specs=[any_spec] * 4,
        out_shape=[jax.ShapeDtypeStruct((FLAT,), jnp.float32)] * 4,
        scratch_shapes=[pltpu.SemaphoreType.DMA],
    )(a_t, b_t, c_t, d_t, tail_flat)


def _sc_scores(user, pos, neg, ui_f, up_f, ii_f, ip_f):
    mesh = plsc.VectorSubcoreMesh(core_axis_name="c", subcore_axis_name="s")

    @functools.partial(
        pl.kernel,
        out_type=[jax.ShapeDtypeStruct((B,), jnp.float32)] * 4,
        mesh=mesh,
        scratch_types=[
            pltpu.VMEM((BPW,), jnp.int32),
            pltpu.VMEM((BPW,), jnp.int32),
            pltpu.VMEM((BPW,), jnp.int32),
            pltpu.VMEM((D, BPW), jnp.float32),
            pltpu.VMEM((D, BPW), jnp.float32),
            pltpu.VMEM((D, BPW), jnp.float32),
            pltpu.VMEM((D, BPW), jnp.float32),
            pltpu.VMEM((D, BPW), jnp.float32),
            pltpu.VMEM((D, BPW), jnp.float32),
            pltpu.VMEM((BPW,), jnp.float32),
            pltpu.VMEM((BPW,), jnp.float32),
            pltpu.VMEM((BPW,), jnp.float32),
            pltpu.VMEM((BPW,), jnp.float32),
            pltpu.SemaphoreType.DMA,
        ],
        compiler_params=pltpu.CompilerParams(
            needs_layout_passes=False, use_tc_tiling_on_sc=False),
    )
    def body(user_h, pos_h, neg_h, uif_h, upf_h, iif_h, ipf_h,
             o_pint, o_nint, o_ppop, o_npop,
             uidx, pidx, nidx, cui, cup, cpi, cpp, cni, cnp,
             s_pint, s_nint, s_ppop, s_npop, sem):
        wid = lax.axis_index("s") * NC + lax.axis_index("c")
        base = wid * BPW

        pltpu.sync_copy(user_h.at[pl.ds(base, BPW)], uidx)
        pltpu.sync_copy(pos_h.at[pl.ds(base, BPW)], pidx)
        pltpu.sync_copy(neg_h.at[pl.ds(base, BPW)], nidx)

        handles = []
        for c in range(D):
            off = c * CPAD
            handles.append(pltpu.async_copy(
                uif_h.at[pl.ds(off, CPAD)].at[uidx], cui.at[c], sem))
            handles.append(pltpu.async_copy(
                upf_h.at[pl.ds(off, CPAD)].at[uidx], cup.at[c], sem))
            handles.append(pltpu.async_copy(
                iif_h.at[pl.ds(off, CPAD)].at[pidx], cpi.at[c], sem))
            handles.append(pltpu.async_copy(
                ipf_h.at[pl.ds(off, CPAD)].at[pidx], cpp.at[c], sem))
            handles.append(pltpu.async_copy(
                iif_h.at[pl.ds(off, CPAD)].at[nidx], cni.at[c], sem))
            handles.append(pltpu.async_copy(
                ipf_h.at[pl.ds(off, CPAD)].at[nidx], cnp.at[c], sem))
        for h in handles:
            h.wait()

        def blk_body(blk, _):
            off = blk * L
            zero = jnp.zeros((L,), jnp.float32)
            a_pint, a_nint, a_ppop, a_npop = zero, zero, zero, zero
            for c in range(D):
                ui = cui[c, pl.ds(off, L)]
                up = cup[c, pl.ds(off, L)]
                pi = cpi[c, pl.ds(off, L)]
                pp = cpp[c, pl.ds(off, L)]
                ni = cni[c, pl.ds(off, L)]
                np_ = cnp[c, pl.ds(off, L)]
                a_pint = a_pint + ui * pi
                a_nint = a_nint + ui * ni
                a_ppop = a_ppop + up * pp
                a_npop = a_npop + up * np_
            s_pint[pl.ds(off, L)] = a_pint
            s_nint[pl.ds(off, L)] = a_nint
            s_ppop[pl.ds(off, L)] = a_ppop
            s_npop[pl.ds(off, L)] = a_npop
            return _

        lax.fori_loop(0, BPW // L, blk_body, None)

        pltpu.sync_copy(s_pint, o_pint.at[pl.ds(base, BPW)])
        pltpu.sync_copy(s_nint, o_nint.at[pl.ds(base, BPW)])
        pltpu.sync_copy(s_ppop, o_ppop.at[pl.ds(base, BPW)])
        pltpu.sync_copy(s_npop, o_npop.at[pl.ds(base, BPW)])

    return body(user, pos, neg, ui_f, up_f, ii_f, ip_f)


def _tc_loss_body(pint_ref, nint_ref, ppop_ref, npop_ref, mask_ref, out_ref):
    m = jnp.clip(mask_ref[...], 0.0, 1.0)

    def bpr(x):
        sig = 1.0 / (1.0 + jnp.exp(-x))
        return -jnp.log(sig + 1e-08)

    pint = pint_ref[...]
    nint = nint_ref[...]
    ppop = ppop_ref[...]
    npop = npop_ref[...]
    total = (
        jnp.sum(bpr(pint - nint) * m)
        + jnp.sum(bpr(npop - ppop) * (1.0 - m))
        + jnp.sum(bpr(ppop - npop) * m)
    )
    out_ref[0, 0] = total / B


def kernel(user, pos, neg, mask, pos_period, neg_period,
           users_int, users_pop, items_int, items_pop):
    del pos_period, neg_period
    tails = jnp.stack([users_int[NMAIN:], users_pop[NMAIN:],
                       items_int[NMAIN:], items_pop[NMAIN:]])
    tail_flat = jnp.pad(jnp.transpose(tails, (0, 2, 1)),
                        ((0, 0), (0, 0), (0, 128 - NTAIL))).reshape(-1)
    ui_f, up_f, ii_f, ip_f = _detile(
        users_int.T, users_pop.T, items_int.T, items_pop.T, tail_flat)

    pint, nint, ppop, npop = _sc_scores(
        user.astype(jnp.int32), pos.astype(jnp.int32), neg.astype(jnp.int32),
        ui_f, up_f, ii_f, ip_f)

    shape2d = (B // 128, 128)
    loss = pl.pallas_call(
        _tc_loss_body,
        out_shape=jax.ShapeDtypeStruct((1, 1), jnp.float32),
        out_specs=pl.BlockSpec(memory_space=pltpu.SMEM),
    )(pint.reshape(shape2d), nint.reshape(shape2d),
      ppop.reshape(shape2d), npop.reshape(shape2d),
      mask.astype(jnp.float32).reshape(shape2d))
    return loss[0, 0]


# speedup vs baseline: 33.8633x; 33.8633x over previous
import functools

import jax
import jax.numpy as jnp
from jax import lax
from jax.experimental import pallas as pl
from jax.experimental.pallas import tpu as pltpu
from jax.experimental.pallas import tpu_sc as plsc

B = 16384
D = 32
NC = 2
NS = 16
L = 16
NW = NC * NS
BPW = B // NW

NROWS = 1000000
NMAIN = 999936
NTAIL = NROWS - NMAIN
CPAD = 1000064
FLAT = D * CPAD


_W = 499968
_NCHUNK = NMAIN // _W


def _detile_body(a_ref, b_ref, c_ref, d_ref, tail_ref,
                 oa_ref, ob_ref, oc_ref, od_ref,
                 buf0, buf1, rs0, rs1, ws0, ws1, tsem):
    tables = ((a_ref, oa_ref), (b_ref, ob_ref), (c_ref, oc_ref),
              (d_ref, od_ref))
    chunks = [(t, cb, j)
              for t in range(4) for cb in range(4) for j in range(_NCHUNK)]
    bufs = (buf0, buf1)
    rsems = (rs0, rs1)
    wsems = (ws0, ws1)

    def read(k):
        t, cb, j = chunks[k]
        src, _ = tables[t]
        return pltpu.async_copy(
            src.at[pl.ds(cb * 8, 8), pl.ds(j * _W, _W)],
            bufs[k % 2], rsems[k % 2])

    def write(k):
        t, cb, j = chunks[k]
        _, dst = tables[t]
        return [
            pltpu.async_copy(
                bufs[k % 2].at[ci],
                dst.at[pl.ds((cb * 8 + ci) * CPAD + j * _W, _W)],
                wsems[k % 2])
            for ci in range(8)
        ]

    n = len(chunks)
    pending_writes = {}
    rh = {0: read(0)}
    for k in range(n):
        if k + 1 < n:
            for h in pending_writes.pop(k - 1, []):
                h.wait()
            rh[k + 1] = read(k + 1)
        rh.pop(k).wait()
        pending_writes[k] = write(k)
    for k, hs in sorted(pending_writes.items()):
        for h in hs:
            h.wait()

    tail_handles = []
    for t, (_, dst) in enumerate(tables):
        for c in range(D):
            tail_handles.append(pltpu.async_copy(
                tail_ref.at[pl.ds((t * D + c) * 128, 128)],
                dst.at[pl.ds(c * CPAD + NMAIN, 128)], tsem))
    for h in tail_handles:
        h.wait()


def _detile(a_t, b_t, c_t, d_t, tail_flat):
    any_spec = pl.BlockSpec(memory_space=pl.ANY)
    return pl.pallas_call(
        _detile_body,
        in_specs=[any_spec] * 5,
        out_specs=[any_spec] * 4,
        out_shape=[jax.ShapeDtypeStruct((FLAT,), jnp.float32)] * 4,
        scratch_shapes=[
            pltpu.VMEM((8, _W), jnp.float32),
            pltpu.VMEM((8, _W), jnp.float32),
            pltpu.SemaphoreType.DMA,
            pltpu.SemaphoreType.DMA,
            pltpu.SemaphoreType.DMA,
            pltpu.SemaphoreType.DMA,
            pltpu.SemaphoreType.DMA,
        ],
    )(a_t, b_t, c_t, d_t, tail_flat)


def _sc_scores(user, pos, neg, ui_f, up_f, ii_f, ip_f):
    mesh = plsc.VectorSubcoreMesh(core_axis_name="c", subcore_axis_name="s")

    @functools.partial(
        pl.kernel,
        out_type=[jax.ShapeDtypeStruct((B,), jnp.float32)] * 4,
        mesh=mesh,
        scratch_types=[
            pltpu.VMEM((BPW,), jnp.int32),
            pltpu.VMEM((BPW,), jnp.int32),
            pltpu.VMEM((BPW,), jnp.int32),
            pltpu.VMEM((D, BPW), jnp.float32),
            pltpu.VMEM((D, BPW), jnp.float32),
            pltpu.VMEM((D, BPW), jnp.float32),
            pltpu.VMEM((D, BPW), jnp.float32),
            pltpu.VMEM((D, BPW), jnp.float32),
            pltpu.VMEM((D, BPW), jnp.float32),
            pltpu.VMEM((BPW,), jnp.float32),
            pltpu.VMEM((BPW,), jnp.float32),
            pltpu.VMEM((BPW,), jnp.float32),
            pltpu.VMEM((BPW,), jnp.float32),
            pltpu.SemaphoreType.DMA,
        ],
        compiler_params=pltpu.CompilerParams(
            needs_layout_passes=False, use_tc_tiling_on_sc=False),
    )
    def body(user_h, pos_h, neg_h, uif_h, upf_h, iif_h, ipf_h,
             o_pint, o_nint, o_ppop, o_npop,
             uidx, pidx, nidx, cui, cup, cpi, cpp, cni, cnp,
             s_pint, s_nint, s_ppop, s_npop, sem):
        wid = lax.axis_index("s") * NC + lax.axis_index("c")
        base = wid * BPW

        pltpu.sync_copy(user_h.at[pl.ds(base, BPW)], uidx)
        pltpu.sync_copy(pos_h.at[pl.ds(base, BPW)], pidx)
        pltpu.sync_copy(neg_h.at[pl.ds(base, BPW)], nidx)

        handles = []
        for c in range(D):
            off = c * CPAD
            handles.append(pltpu.async_copy(
                uif_h.at[pl.ds(off, CPAD)].at[uidx], cui.at[c], sem))
            handles.append(pltpu.async_copy(
                upf_h.at[pl.ds(off, CPAD)].at[uidx], cup.at[c], sem))
            handles.append(pltpu.async_copy(
                iif_h.at[pl.ds(off, CPAD)].at[pidx], cpi.at[c], sem))
            handles.append(pltpu.async_copy(
                ipf_h.at[pl.ds(off, CPAD)].at[pidx], cpp.at[c], sem))
            handles.append(pltpu.async_copy(
                iif_h.at[pl.ds(off, CPAD)].at[nidx], cni.at[c], sem))
            handles.append(pltpu.async_copy(
                ipf_h.at[pl.ds(off, CPAD)].at[nidx], cnp.at[c], sem))
        for h in handles:
            h.wait()

        def blk_body(blk, _):
            off = blk * L
            zero = jnp.zeros((L,), jnp.float32)
            a_pint, a_nint, a_ppop, a_npop = zero, zero, zero, zero
            for c in range(D):
                ui = cui[c, pl.ds(off, L)]
                up = cup[c, pl.ds(off, L)]
                pi = cpi[c, pl.ds(off, L)]
                pp = cpp[c, pl.ds(off, L)]
                ni = cni[c, pl.ds(off, L)]
                np_ = cnp[c, pl.ds(off, L)]
                a_pint = a_pint + ui * pi
                a_nint = a_nint + ui * ni
                a_ppop = a_ppop + up * pp
                a_npop = a_npop + up * np_
            s_pint[pl.ds(off, L)] = a_pint
            s_nint[pl.ds(off, L)] = a_nint
            s_ppop[pl.ds(off, L)] = a_ppop
            s_npop[pl.ds(off, L)] = a_npop
            return _

        lax.fori_loop(0, BPW // L, blk_body, None)

        pltpu.sync_copy(s_pint, o_pint.at[pl.ds(base, BPW)])
        pltpu.sync_copy(s_nint, o_nint.at[pl.ds(base, BPW)])
        pltpu.sync_copy(s_ppop, o_ppop.at[pl.ds(base, BPW)])
        pltpu.sync_copy(s_npop, o_npop.at[pl.ds(base, BPW)])

    return body(user, pos, neg, ui_f, up_f, ii_f, ip_f)


def _tc_loss_body(pint_ref, nint_ref, ppop_ref, npop_ref, mask_ref, out_ref):
    m = jnp.clip(mask_ref[...], 0.0, 1.0)

    def bpr(x):
        sig = 1.0 / (1.0 + jnp.exp(-x))
        return -jnp.log(sig + 1e-08)

    pint = pint_ref[...]
    nint = nint_ref[...]
    ppop = ppop_ref[...]
    npop = npop_ref[...]
    total = (
        jnp.sum(bpr(pint - nint) * m)
        + jnp.sum(bpr(npop - ppop) * (1.0 - m))
        + jnp.sum(bpr(ppop - npop) * m)
    )
    out_ref[0, 0] = total / B


def kernel(user, pos, neg, mask, pos_period, neg_period,
           users_int, users_pop, items_int, items_pop):
    del pos_period, neg_period
    tails = jnp.stack([users_int[NMAIN:], users_pop[NMAIN:],
                       items_int[NMAIN:], items_pop[NMAIN:]])
    tail_flat = jnp.pad(jnp.transpose(tails, (0, 2, 1)),
                        ((0, 0), (0, 0), (0, 128 - NTAIL))).reshape(-1)
    ui_f, up_f, ii_f, ip_f = _detile(
        users_int.T, users_pop.T, items_int.T, items_pop.T, tail_flat)

    pint, nint, ppop, npop = _sc_scores(
        user.astype(jnp.int32), pos.astype(jnp.int32), neg.astype(jnp.int32),
        ui_f, up_f, ii_f, ip_f)

    shape2d = (B // 128, 128)
    loss = pl.pallas_call(
        _tc_loss_body,
        out_shape=jax.ShapeDtypeStruct((1, 1), jnp.float32),
        out_specs=pl.BlockSpec(memory_space=pltpu.SMEM),
    )(pint.reshape(shape2d), nint.reshape(shape2d),
      ppop.reshape(shape2d), npop.reshape(shape2d),
      mask.astype(jnp.float32).reshape(shape2d))
    return loss[0, 0]


# speedup vs baseline: 34.2172x; 1.0105x over previous
import functools

import jax
import jax.numpy as jnp
from jax import lax
from jax.experimental import pallas as pl
from jax.experimental.pallas import tpu as pltpu
from jax.experimental.pallas import tpu_sc as plsc

B = 16384
D = 32
NC = 2
NS = 16
L = 16
NW = NC * NS
BPW = B // NW

NROWS = 1000000
NMAIN = 999936
NTAIL = NROWS - NMAIN
CPAD = 1000064
FLAT = D * CPAD


_W = 499968
_NCHUNK = NMAIN // _W


def _detile_body(a_ref, b_ref, c_ref, d_ref, tail_ref,
                 oa_ref, ob_ref, oc_ref, od_ref,
                 buf0, buf1, rs0, rs1, ws0, ws1, tsem):
    tables = ((a_ref, oa_ref), (b_ref, ob_ref), (c_ref, oc_ref),
              (d_ref, od_ref))
    chunks = [(t, cb, j)
              for t in range(4) for cb in range(4) for j in range(_NCHUNK)]
    bufs = (buf0, buf1)
    rsems = (rs0, rs1)
    wsems = (ws0, ws1)

    def read(k):
        t, cb, j = chunks[k]
        src, _ = tables[t]
        return pltpu.async_copy(
            src.at[pl.ds(cb * 8, 8), pl.ds(j * _W, _W)],
            bufs[k % 2], rsems[k % 2])

    def write(k):
        t, cb, j = chunks[k]
        _, dst = tables[t]
        return [
            pltpu.async_copy(
                bufs[k % 2].at[ci],
                dst.at[pl.ds((cb * 8 + ci) * CPAD + j * _W, _W)],
                wsems[k % 2])
            for ci in range(8)
        ]

    tail_handles = []
    for t, (_, dst) in enumerate(tables):
        for c in range(D):
            tail_handles.append(pltpu.async_copy(
                tail_ref.at[pl.ds((t * D + c) * 128, 128)],
                dst.at[pl.ds(c * CPAD + NMAIN, 128)], tsem))

    n = len(chunks)
    pending_writes = {}
    rh = {0: read(0)}
    for k in range(n):
        if k + 1 < n:
            for h in pending_writes.pop(k - 1, []):
                h.wait()
            rh[k + 1] = read(k + 1)
        rh.pop(k).wait()
        pending_writes[k] = write(k)
    for k, hs in sorted(pending_writes.items()):
        for h in hs:
            h.wait()
    for h in tail_handles:
        h.wait()


def _detile(a_t, b_t, c_t, d_t, tail_flat):
    any_spec = pl.BlockSpec(memory_space=pl.ANY)
    return pl.pallas_call(
        _detile_body,
        in_specs=[any_spec] * 5,
        out_specs=[any_spec] * 4,
        out_shape=[jax.ShapeDtypeStruct((FLAT,), jnp.float32)] * 4,
        scratch_shapes=[
            pltpu.VMEM((8, _W), jnp.float32),
            pltpu.VMEM((8, _W), jnp.float32),
            pltpu.SemaphoreType.DMA,
            pltpu.SemaphoreType.DMA,
            pltpu.SemaphoreType.DMA,
            pltpu.SemaphoreType.DMA,
            pltpu.SemaphoreType.DMA,
        ],
    )(a_t, b_t, c_t, d_t, tail_flat)


def _sc_scores(user, pos, neg, ui_f, up_f, ii_f, ip_f):
    mesh = plsc.VectorSubcoreMesh(core_axis_name="c", subcore_axis_name="s")

    @functools.partial(
        pl.kernel,
        out_type=[jax.ShapeDtypeStruct((B,), jnp.float32)] * 4,
        mesh=mesh,
        scratch_types=[
            pltpu.VMEM((BPW,), jnp.int32),
            pltpu.VMEM((BPW,), jnp.int32),
            pltpu.VMEM((BPW,), jnp.int32),
            pltpu.VMEM((D, BPW), jnp.float32),
            pltpu.VMEM((D, BPW), jnp.float32),
            pltpu.VMEM((D, BPW), jnp.float32),
            pltpu.VMEM((D, BPW), jnp.float32),
            pltpu.VMEM((D, BPW), jnp.float32),
            pltpu.VMEM((D, BPW), jnp.float32),
            pltpu.VMEM((BPW,), jnp.float32),
            pltpu.VMEM((BPW,), jnp.float32),
            pltpu.VMEM((BPW,), jnp.float32),
            pltpu.VMEM((BPW,), jnp.float32),
            pltpu.SemaphoreType.DMA,
            pltpu.SemaphoreType.DMA,
            pltpu.SemaphoreType.DMA,
            pltpu.SemaphoreType.DMA,
        ],
        compiler_params=pltpu.CompilerParams(
            needs_layout_passes=False, use_tc_tiling_on_sc=False),
    )
    def body(user_h, pos_h, neg_h, uif_h, upf_h, iif_h, ipf_h,
             o_pint, o_nint, o_ppop, o_npop,
             uidx, pidx, nidx, cui, cup, cpi, cpp, cni, cnp,
             s_pint, s_nint, s_ppop, s_npop, sem0, sem1, sem2, sem3):
        wid = lax.axis_index("s") * NC + lax.axis_index("c")
        base = wid * BPW
        sems = (sem0, sem1, sem2, sem3)
        cg = D // len(sems)

        pltpu.sync_copy(user_h.at[pl.ds(base, BPW)], uidx)
        pltpu.sync_copy(pos_h.at[pl.ds(base, BPW)], pidx)
        pltpu.sync_copy(neg_h.at[pl.ds(base, BPW)], nidx)

        handles = [[] for _ in sems]
        for c in range(D):
            g = c // cg
            off = c * CPAD
            sem = sems[g]
            handles[g].append(pltpu.async_copy(
                uif_h.at[pl.ds(off, CPAD)].at[uidx], cui.at[c], sem))
            handles[g].append(pltpu.async_copy(
                upf_h.at[pl.ds(off, CPAD)].at[uidx], cup.at[c], sem))
            handles[g].append(pltpu.async_copy(
                iif_h.at[pl.ds(off, CPAD)].at[pidx], cpi.at[c], sem))
            handles[g].append(pltpu.async_copy(
                ipf_h.at[pl.ds(off, CPAD)].at[pidx], cpp.at[c], sem))
            handles[g].append(pltpu.async_copy(
                iif_h.at[pl.ds(off, CPAD)].at[nidx], cni.at[c], sem))
            handles[g].append(pltpu.async_copy(
                ipf_h.at[pl.ds(off, CPAD)].at[nidx], cnp.at[c], sem))

        for g in range(len(sems)):
            for h in handles[g]:
                h.wait()

            def blk_body(blk, _, g=g):
                off = blk * L
                zero = jnp.zeros((L,), jnp.float32)
                a_pint, a_nint, a_ppop, a_npop = zero, zero, zero, zero
                for c in range(g * cg, (g + 1) * cg):
                    ui = cui[c, pl.ds(off, L)]
                    up = cup[c, pl.ds(off, L)]
                    pi = cpi[c, pl.ds(off, L)]
                    pp = cpp[c, pl.ds(off, L)]
                    ni = cni[c, pl.ds(off, L)]
                    np_ = cnp[c, pl.ds(off, L)]
                    a_pint = a_pint + ui * pi
                    a_nint = a_nint + ui * ni
                    a_ppop = a_ppop + up * pp
                    a_npop = a_npop + up * np_
                if g > 0:
                    a_pint = a_pint + s_pint[pl.ds(off, L)]
                    a_nint = a_nint + s_nint[pl.ds(off, L)]
                    a_ppop = a_ppop + s_ppop[pl.ds(off, L)]
                    a_npop = a_npop + s_npop[pl.ds(off, L)]
                s_pint[pl.ds(off, L)] = a_pint
                s_nint[pl.ds(off, L)] = a_nint
                s_ppop[pl.ds(off, L)] = a_ppop
                s_npop[pl.ds(off, L)] = a_npop
                return _

            lax.fori_loop(0, BPW // L, blk_body, None)

        pltpu.sync_copy(s_pint, o_pint.at[pl.ds(base, BPW)])
        pltpu.sync_copy(s_nint, o_nint.at[pl.ds(base, BPW)])
        pltpu.sync_copy(s_ppop, o_ppop.at[pl.ds(base, BPW)])
        pltpu.sync_copy(s_npop, o_npop.at[pl.ds(base, BPW)])

    return body(user, pos, neg, ui_f, up_f, ii_f, ip_f)


def _tc_loss_body(pint_ref, nint_ref, ppop_ref, npop_ref, mask_ref, out_ref):
    m = jnp.clip(mask_ref[...], 0.0, 1.0)

    def bpr(x):
        sig = 1.0 / (1.0 + jnp.exp(-x))
        return -jnp.log(sig + 1e-08)

    pint = pint_ref[...]
    nint = nint_ref[...]
    ppop = ppop_ref[...]
    npop = npop_ref[...]
    total = (
        jnp.sum(bpr(pint - nint) * m)
        + jnp.sum(bpr(npop - ppop) * (1.0 - m))
        + jnp.sum(bpr(ppop - npop) * m)
    )
    out_ref[0, 0] = total / B


def kernel(user, pos, neg, mask, pos_period, neg_period,
           users_int, users_pop, items_int, items_pop):
    del pos_period, neg_period
    tails = jnp.stack([users_int[NMAIN:], users_pop[NMAIN:],
                       items_int[NMAIN:], items_pop[NMAIN:]])
    tail_flat = jnp.pad(jnp.transpose(tails, (0, 2, 1)),
                        ((0, 0), (0, 0), (0, 128 - NTAIL))).reshape(-1)
    ui_f, up_f, ii_f, ip_f = _detile(
        users_int.T, users_pop.T, items_int.T, items_pop.T, tail_flat)

    pint, nint, ppop, npop = _sc_scores(
        user.astype(jnp.int32), pos.astype(jnp.int32), neg.astype(jnp.int32),
        ui_f, up_f, ii_f, ip_f)

    shape2d = (B // 128, 128)
    loss = pl.pallas_call(
        _tc_loss_body,
        out_shape=jax.ShapeDtypeStruct((1, 1), jnp.float32),
        out_specs=pl.BlockSpec(memory_space=pltpu.SMEM),
    )(pint.reshape(shape2d), nint.reshape(shape2d),
      ppop.reshape(shape2d), npop.reshape(shape2d),
      mask.astype(jnp.float32).reshape(shape2d))
    return loss[0, 0]


# speedup vs baseline: 35.2348x; 1.0297x over previous
import functools

import jax
import jax.numpy as jnp
from jax import lax
from jax.experimental import pallas as pl
from jax.experimental.pallas import tpu as pltpu
from jax.experimental.pallas import tpu_sc as plsc

B = 16384
D = 32
NC = 2
NS = 16
L = 16
NW = NC * NS
BPW = B // NW

NROWS = 1000000
NMAIN = 999936
NTAIL = NROWS - NMAIN
CPAD = 1000064
FLAT = D * CPAD

_W = 499968
_NCHUNK = NMAIN // _W


def _detile_body(src_ref, tail_ref, dst_ref, buf0, buf1, rs0, rs1,
                 ws0, ws1, tsem):
    chunks = [(cb, j) for cb in range(4) for j in range(_NCHUNK)]
    bufs = (buf0, buf1)
    rsems = (rs0, rs1)
    wsems = (ws0, ws1)

    tail_handles = []
    for c in range(D):
        tail_handles.append(pltpu.async_copy(
            tail_ref.at[pl.ds(c * 128, 128)],
            dst_ref.at[pl.ds(c * CPAD + NMAIN, 128)], tsem))

    def read(k):
        cb, j = chunks[k]
        return pltpu.async_copy(
            src_ref.at[pl.ds(cb * 8, 8), pl.ds(j * _W, _W)],
            bufs[k % 2], rsems[k % 2])

    def write(k):
        cb, j = chunks[k]
        return [
            pltpu.async_copy(
                bufs[k % 2].at[ci],
                dst_ref.at[pl.ds((cb * 8 + ci) * CPAD + j * _W, _W)],
                wsems[k % 2])
            for ci in range(8)
        ]

    n = len(chunks)
    pending_writes = {}
    rh = {0: read(0)}
    for k in range(n):
        if k + 1 < n:
            for h in pending_writes.pop(k - 1, []):
                h.wait()
            rh[k + 1] = read(k + 1)
        rh.pop(k).wait()
        pending_writes[k] = write(k)
    for k, hs in sorted(pending_writes.items()):
        for h in hs:
            h.wait()
    for h in tail_handles:
        h.wait()


def _detile(table_t, tail_flat):
    any_spec = pl.BlockSpec(memory_space=pl.ANY)
    return pl.pallas_call(
        _detile_body,
        in_specs=[any_spec] * 2,
        out_specs=any_spec,
        out_shape=jax.ShapeDtypeStruct((FLAT,), jnp.float32),
        scratch_shapes=[
            pltpu.VMEM((8, _W), jnp.float32),
            pltpu.VMEM((8, _W), jnp.float32),
            pltpu.SemaphoreType.DMA,
            pltpu.SemaphoreType.DMA,
            pltpu.SemaphoreType.DMA,
            pltpu.SemaphoreType.DMA,
            pltpu.SemaphoreType.DMA,
        ],
    )(table_t, tail_flat)


_SC_PARAMS = pltpu.CompilerParams(
    needs_layout_passes=False, use_tc_tiling_on_sc=False)


def _sc_gather1(flat, idx):
    mesh = plsc.VectorSubcoreMesh(core_axis_name="c", subcore_axis_name="s")

    @functools.partial(
        pl.kernel,
        out_type=jax.ShapeDtypeStruct((D * B,), jnp.float32),
        mesh=mesh,
        scratch_types=[
            pltpu.VMEM((BPW,), jnp.int32),
            pltpu.VMEM((D, BPW), jnp.float32),
            pltpu.SemaphoreType.DMA,
        ],
        compiler_params=_SC_PARAMS,
    )
    def body(idx_h, flat_h, out_h, vidx, cols, sem):
        wid = lax.axis_index("s") * NC + lax.axis_index("c")
        base = wid * BPW
        pltpu.sync_copy(idx_h.at[pl.ds(base, BPW)], vidx)
        handles = []
        for c in range(D):
            handles.append(pltpu.async_copy(
                flat_h.at[pl.ds(c * CPAD, CPAD)].at[vidx], cols.at[c], sem))
        for c in range(D):
            handles[c].wait()
            pltpu.sync_copy(cols.at[c], out_h.at[pl.ds(c * B + base, BPW)])

    return body(idx, flat)


def _sc_gather2(flat, idx_a, idx_b):
    mesh = plsc.VectorSubcoreMesh(core_axis_name="c", subcore_axis_name="s")

    @functools.partial(
        pl.kernel,
        out_type=[jax.ShapeDtypeStruct((D * B,), jnp.float32)] * 2,
        mesh=mesh,
        scratch_types=[
            pltpu.VMEM((BPW,), jnp.int32),
            pltpu.VMEM((BPW,), jnp.int32),
            pltpu.VMEM((D, BPW), jnp.float32),
            pltpu.VMEM((D, BPW), jnp.float32),
            pltpu.SemaphoreType.DMA,
        ],
        compiler_params=_SC_PARAMS,
    )
    def body(ia_h, ib_h, flat_h, oa_h, ob_h, via, vib, ca, cb, sem):
        wid = lax.axis_index("s") * NC + lax.axis_index("c")
        base = wid * BPW
        pltpu.sync_copy(ia_h.at[pl.ds(base, BPW)], via)
        pltpu.sync_copy(ib_h.at[pl.ds(base, BPW)], vib)
        handles = []
        for c in range(D):
            src = flat_h.at[pl.ds(c * CPAD, CPAD)]
            handles.append(pltpu.async_copy(src.at[via], ca.at[c], sem))
            handles.append(pltpu.async_copy(src.at[vib], cb.at[c], sem))
        for c in range(D):
            handles[2 * c].wait()
            handles[2 * c + 1].wait()
            pltpu.sync_copy(ca.at[c], oa_h.at[pl.ds(c * B + base, BPW)])
            pltpu.sync_copy(cb.at[c], ob_h.at[pl.ds(c * B + base, BPW)])

    return body(idx_a, idx_b, flat)


def _tc_loss_body(ui_ref, up_ref, pi_ref, pp_ref, ni_ref, np_ref,
                  mask_ref, out_ref):
    shape2d = (B // 128, 128)

    def dot(a_ref, b_ref):
        acc = jnp.zeros((B,), jnp.float32)
        for c in range(D):
            acc = acc + a_ref[pl.ds(c * B, B)] * b_ref[pl.ds(c * B, B)]
        return acc.reshape(shape2d)

    pint = dot(ui_ref, pi_ref)
    nint = dot(ui_ref, ni_ref)
    ppop = dot(up_ref, pp_ref)
    npop = dot(up_ref, np_ref)
    m = jnp.clip(mask_ref[...], 0.0, 1.0).reshape(shape2d)

    def bpr(x):
        sig = 1.0 / (1.0 + jnp.exp(-x))
        return -jnp.log(sig + 1e-08)

    total = (
        jnp.sum(bpr(pint - nint) * m)
        + jnp.sum(bpr(npop - ppop) * (1.0 - m))
        + jnp.sum(bpr(ppop - npop) * m)
    )
    out_ref[0, 0] = total / B


def kernel(user, pos, neg, mask, pos_period, neg_period,
           users_int, users_pop, items_int, items_pop):
    del pos_period, neg_period
    user = user.astype(jnp.int32)
    pos = pos.astype(jnp.int32)
    neg = neg.astype(jnp.int32)

    tails = jnp.stack([items_int[NMAIN:], items_pop[NMAIN:],
                       users_int[NMAIN:], users_pop[NMAIN:]])
    tail4 = jnp.pad(jnp.transpose(tails, (0, 2, 1)),
                    ((0, 0), (0, 0), (0, 128 - NTAIL))).reshape(4, -1)

    ii_f = _detile(items_int.T, tail4[0])
    g_pi, g_ni = _sc_gather2(ii_f, pos, neg)
    ip_f = _detile(items_pop.T, tail4[1])
    g_pp, g_np = _sc_gather2(ip_f, pos, neg)
    ui_f = _detile(users_int.T, tail4[2])
    g_ui = _sc_gather1(ui_f, user)
    up_f = _detile(users_pop.T, tail4[3])
    g_up = _sc_gather1(up_f, user)

    loss = pl.pallas_call(
        _tc_loss_body,
        out_shape=jax.ShapeDtypeStruct((1, 1), jnp.float32),
        out_specs=pl.BlockSpec(memory_space=pltpu.SMEM),
    )(g_ui, g_up, g_pi, g_pp, g_ni, g_np, mask.astype(jnp.float32))
    return loss[0, 0]


# speedup vs baseline: 35.6809x; 1.0127x over previous
import functools

import jax
import jax.numpy as jnp
from jax import lax
from jax.experimental import pallas as pl
from jax.experimental.pallas import tpu as pltpu
from jax.experimental.pallas import tpu_sc as plsc

B = 16384
D = 32
NC = 2
NS = 16
L = 16
NW = NC * NS
BPW = B // NW

NROWS = 1000000
NMAIN = 999936
NTAIL = NROWS - NMAIN
CPAD = 1000064
FLAT = D * CPAD

_W = 499968
_NCHUNK = NMAIN // _W


def _detile_body(src_ref, tail_ref, dst_ref, buf0, buf1, rs0, rs1,
                 ws0, ws1, tsem):
    chunks = [(cb, j) for cb in range(4) for j in range(_NCHUNK)]
    bufs = (buf0, buf1)
    rsems = (rs0, rs1)
    wsems = (ws0, ws1)

    tail_handles = []
    for c in range(D):
        tail_handles.append(pltpu.async_copy(
            tail_ref.at[pl.ds(c * 128, 128)],
            dst_ref.at[pl.ds(c * CPAD + NMAIN, 128)], tsem))

    def read(k):
        cb, j = chunks[k]
        return pltpu.async_copy(
            src_ref.at[pl.ds(cb * 8, 8), pl.ds(j * _W, _W)],
            bufs[k % 2], rsems[k % 2])

    def write(k):
        cb, j = chunks[k]
        return [
            pltpu.async_copy(
                bufs[k % 2].at[ci],
                dst_ref.at[pl.ds((cb * 8 + ci) * CPAD + j * _W, _W)],
                wsems[k % 2])
            for ci in range(8)
        ]

    n = len(chunks)
    pending_writes = {}
    rh = {0: read(0)}
    for k in range(n):
        if k + 1 < n:
            for h in pending_writes.pop(k - 1, []):
                h.wait()
            rh[k + 1] = read(k + 1)
        rh.pop(k).wait()
        pending_writes[k] = write(k)
    for k, hs in sorted(pending_writes.items()):
        for h in hs:
            h.wait()
    for h in tail_handles:
        h.wait()


def _detile(table_t, tail_flat):
    any_spec = pl.BlockSpec(memory_space=pl.ANY)
    return pl.pallas_call(
        _detile_body,
        in_specs=[any_spec] * 2,
        out_specs=any_spec,
        out_shape=jax.ShapeDtypeStruct((FLAT,), jnp.float32),
        scratch_shapes=[
            pltpu.VMEM((8, _W), jnp.float32),
            pltpu.VMEM((8, _W), jnp.float32),
            pltpu.SemaphoreType.DMA,
            pltpu.SemaphoreType.DMA,
            pltpu.SemaphoreType.DMA,
            pltpu.SemaphoreType.DMA,
            pltpu.SemaphoreType.DMA,
        ],
    )(table_t, tail_flat)


_SC_PARAMS = pltpu.CompilerParams(
    needs_layout_passes=False, use_tc_tiling_on_sc=False)


def _sc_gather1(flat, idx):
    mesh = plsc.VectorSubcoreMesh(core_axis_name="c", subcore_axis_name="s")

    @functools.partial(
        pl.kernel,
        out_type=jax.ShapeDtypeStruct((D * B,), jnp.float32),
        mesh=mesh,
        scratch_types=[
            pltpu.VMEM((BPW,), jnp.int32),
            pltpu.VMEM((D, BPW), jnp.float32),
            pltpu.SemaphoreType.DMA,
            pltpu.SemaphoreType.DMA,
        ],
        compiler_params=_SC_PARAMS,
    )
    def body(idx_h, flat_h, out_h, vidx, cols, sem, osem):
        wid = lax.axis_index("s") * NC + lax.axis_index("c")
        base = wid * BPW
        pltpu.sync_copy(idx_h.at[pl.ds(base, BPW)], vidx)
        handles = []
        for c in range(D):
            handles.append(pltpu.async_copy(
                flat_h.at[pl.ds(c * CPAD, CPAD)].at[vidx], cols.at[c], sem))
        out_handles = []
        for c in range(D):
            handles[c].wait()
            out_handles.append(pltpu.async_copy(
                cols.at[c], out_h.at[pl.ds(c * B + base, BPW)], osem))
        for h in out_handles:
            h.wait()

    return body(idx, flat)


def _sc_gather2(flat, idx_a, idx_b):
    mesh = plsc.VectorSubcoreMesh(core_axis_name="c", subcore_axis_name="s")

    @functools.partial(
        pl.kernel,
        out_type=[jax.ShapeDtypeStruct((D * B,), jnp.float32)] * 2,
        mesh=mesh,
        scratch_types=[
            pltpu.VMEM((BPW,), jnp.int32),
            pltpu.VMEM((BPW,), jnp.int32),
            pltpu.VMEM((D, BPW), jnp.float32),
            pltpu.VMEM((D, BPW), jnp.float32),
            pltpu.SemaphoreType.DMA,
            pltpu.SemaphoreType.DMA,
        ],
        compiler_params=_SC_PARAMS,
    )
    def body(ia_h, ib_h, flat_h, oa_h, ob_h, via, vib, ca, cb, sem, osem):
        wid = lax.axis_index("s") * NC + lax.axis_index("c")
        base = wid * BPW
        pltpu.sync_copy(ia_h.at[pl.ds(base, BPW)], via)
        pltpu.sync_copy(ib_h.at[pl.ds(base, BPW)], vib)
        handles = []
        for c in range(D):
            src = flat_h.at[pl.ds(c * CPAD, CPAD)]
            handles.append(pltpu.async_copy(src.at[via], ca.at[c], sem))
            handles.append(pltpu.async_copy(src.at[vib], cb.at[c], sem))
        out_handles = []
        for c in range(D):
            handles[2 * c].wait()
            handles[2 * c + 1].wait()
            out_handles.append(pltpu.async_copy(
                ca.at[c], oa_h.at[pl.ds(c * B + base, BPW)], osem))
            out_handles.append(pltpu.async_copy(
                cb.at[c], ob_h.at[pl.ds(c * B + base, BPW)], osem))
        for h in out_handles:
            h.wait()

    return body(idx_a, idx_b, flat)


def _tc_loss_body(ui_ref, up_ref, pi_ref, pp_ref, ni_ref, np_ref,
                  mask_ref, out_ref):
    shape2d = (B // 128, 128)

    def dot(a_ref, b_ref):
        acc = jnp.zeros((B,), jnp.float32)
        for c in range(D):
            acc = acc + a_ref[pl.ds(c * B, B)] * b_ref[pl.ds(c * B, B)]
        return acc.reshape(shape2d)

    pint = dot(ui_ref, pi_ref)
    nint = dot(ui_ref, ni_ref)
    ppop = dot(up_ref, pp_ref)
    npop = dot(up_ref, np_ref)
    m = jnp.clip(mask_ref[...], 0.0, 1.0).reshape(shape2d)

    def bpr(x):
        sig = 1.0 / (1.0 + jnp.exp(-x))
        return -jnp.log(sig + 1e-08)

    total = (
        jnp.sum(bpr(pint - nint) * m)
        + jnp.sum(bpr(npop - ppop) * (1.0 - m))
        + jnp.sum(bpr(ppop - npop) * m)
    )
    out_ref[0, 0] = total / B


def kernel(user, pos, neg, mask, pos_period, neg_period,
           users_int, users_pop, items_int, items_pop):
    del pos_period, neg_period
    user = user.astype(jnp.int32)
    pos = pos.astype(jnp.int32)
    neg = neg.astype(jnp.int32)

    tails = jnp.stack([items_int[NMAIN:], items_pop[NMAIN:],
                       users_int[NMAIN:], users_pop[NMAIN:]])
    tail4 = jnp.pad(jnp.transpose(tails, (0, 2, 1)),
                    ((0, 0), (0, 0), (0, 128 - NTAIL))).reshape(4, -1)

    ii_f = _detile(items_int.T, tail4[0])
    g_pi, g_ni = _sc_gather2(ii_f, pos, neg)
    ip_f = _detile(items_pop.T, tail4[1])
    g_pp, g_np = _sc_gather2(ip_f, pos, neg)
    ui_f = _detile(users_int.T, tail4[2])
    g_ui = _sc_gather1(ui_f, user)
    up_f = _detile(users_pop.T, tail4[3])
    g_up = _sc_gather1(up_f, user)

    loss = pl.pallas_call(
        _tc_loss_body,
        out_shape=jax.ShapeDtypeStruct((1, 1), jnp.float32),
        out_specs=pl.BlockSpec(memory_space=pltpu.SMEM),
    )(g_ui, g_up, g_pi, g_pp, g_ni, g_np, mask.astype(jnp.float32))
    return loss[0, 0]
